# Initial kernel scaffold; baseline (speedup 1.0000x reference)
import jax
import jax.numpy as jnp
from jax.experimental import pallas as pl


def kernel(coord_v):
    raise NotImplementedError("write your pallas kernel here")


# speedup vs baseline: 1.0313x; 1.0313x over previous
import functools

import jax
import jax.numpy as jnp
from jax import lax
from jax.experimental import pallas as pl
from jax.experimental.pallas import tpu as pltpu
from jax.experimental.pallas import tpu_sc as plsc

NB = 128
NP = 8192
G = 64
CELLS = G * G * G
NC, NS, L = 2, 16, 16
NW = NC * NS
BPW = NB // NW
ZCHUNK = 32768
NZ = CELLS // ZCHUNK
ROWS = NP // 128


def _body(coords_hbm, out_hbm, zeros_v, coords_v, idx_v, ones_v, zsem, ssem):
    wid = lax.axis_index("s") * NC + lax.axis_index("c")
    lanes = lax.iota(jnp.int32, L)

    def fill_ones(i, _):
        ones_v[pl.ds(i * L, L)] = jnp.full((L,), 1.0, jnp.float32)
        return 0
    lax.fori_loop(0, 128 // L, fill_ones, 0)

    def fill_zero(i, _):
        zeros_v[pl.ds(i * L, L)] = jnp.zeros((L,), jnp.float32)
        return 0
    lax.fori_loop(0, ZCHUNK // L, fill_zero, 0)

    zcopies = [
        pltpu.async_copy(
            zeros_v,
            out_hbm.at[pl.ds((wid * BPW + bl) * CELLS + z * ZCHUNK, ZCHUNK)],
            zsem)
        for bl in range(BPW)
        for z in range(NZ)
    ]

    for bl in range(BPW):
        b = wid * BPW + bl
        base = b * CELLS
        pltpu.sync_copy(coords_hbm.at[pl.ds(b * NP * 3, NP * 3)], coords_v)

        def idx_row(j, _):
            for k in range(8):
                p3 = (j * 128 + k * L + lanes) * 3
                x = plsc.load_gather(coords_v, [p3])
                y = plsc.load_gather(coords_v, [p3 + 1])
                z = plsc.load_gather(coords_v, [p3 + 2])
                ix = (x * 64.0).astype(jnp.int32)
                iy = (y * 64.0).astype(jnp.int32)
                iz = (z * 64.0).astype(jnp.int32)
                idx_v[bl * ROWS + j, pl.ds(k * L, L)] = (
                    base + ix * 4096 + iz * 64 + iy)
            return 0
        lax.fori_loop(0, ROWS, idx_row, 0)

    for c in zcopies:
        c.wait()
    scopies = [
        pltpu.async_copy(ones_v, out_hbm.at[idx_v.at[r]], ssem)
        for r in range(BPW * ROWS)
    ]
    for c in scopies:
        c.wait()


_mesh = plsc.VectorSubcoreMesh(core_axis_name="c", subcore_axis_name="s")

_scatter = functools.partial(
    pl.kernel,
    out_type=jax.ShapeDtypeStruct((NB * CELLS,), jnp.float32),
    mesh=_mesh,
    scratch_types=[
        pltpu.VMEM((ZCHUNK,), jnp.float32),
        pltpu.VMEM((NP * 3,), jnp.float32),
        pltpu.VMEM((BPW * ROWS, 128), jnp.int32),
        pltpu.VMEM((128,), jnp.float32),
        pltpu.SemaphoreType.DMA,
        pltpu.SemaphoreType.DMA,
    ],
    compiler_params=pltpu.CompilerParams(needs_layout_passes=False),
)(_body)


def kernel(coord_v):
    out = _scatter(coord_v.reshape(NB * NP * 3))
    return out.reshape(NB, G, G, G)
